# baseline (speedup 1.0000x reference)
import jax
import jax.numpy as jnp
from jax import lax
from jax.experimental import pallas as pl
from jax.experimental.pallas import tpu as pltpu
from jax.experimental.pallas import tpu_sc as plsc

_START, _END = 512, 2560
_CHUNK = 32


def _sc_zero_body(o_hbm, zb, sz):
    nbatch, seq, feat = o_hbm.shape
    nc = 2
    wid = lax.axis_index("s") * nc + lax.axis_index("c")
    nw = 32
    n_pad = _START + (seq - _END)
    z_rows = nbatch * n_pad // nw
    per_b = n_pad // z_rows
    b = wid // per_b
    z_off_raw = (wid % per_b) * z_rows
    z_off = jnp.where(
        z_off_raw < _START, z_off_raw, _END + (z_off_raw - _START)
    )

    def zrow(i, c):
        for j in range(feat // 16):
            zb[i, pl.ds(j * 16, 16)] = jnp.zeros((16,), jnp.float32)
        return c

    lax.fori_loop(0, _CHUNK, zrow, 0)

    zdmas = [
        pltpu.make_async_copy(
            zb, o_hbm.at[b, pl.ds(z_off + c * _CHUNK, _CHUNK)], sz
        )
        for c in range(z_rows // _CHUNK)
    ]
    for d in zdmas:
        d.start()
    for d in zdmas:
        d.wait()


def _tc_window_body(g_ref, z_ref, o_ref):
    del z_ref
    o_ref[...] = g_ref[...]


def kernel(grad_last, input):
    b0, b1, g_rows, feat = grad_last.shape
    seq = input.shape[1]
    nb = b0 * b1
    g = grad_last.reshape(nb, g_rows, feat)

    mesh = plsc.VectorSubcoreMesh(core_axis_name="c", subcore_axis_name="s")
    padded = pl.kernel(
        _sc_zero_body,
        out_type=jax.ShapeDtypeStruct((nb, seq, feat), grad_last.dtype),
        mesh=mesh,
        scratch_types=[
            pltpu.VMEM((_CHUNK, feat), jnp.float32),
            pltpu.SemaphoreType.DMA,
        ],
    )()

    blk = 512
    n_wblocks = g_rows // blk
    lo = _START // blk
    out = pl.pallas_call(
        _tc_window_body,
        grid=(nb, n_wblocks),
        in_specs=[
            pl.BlockSpec((1, blk, feat), lambda b, j: (b, j, 0)),
            pl.BlockSpec(memory_space=pltpu.MemorySpace.HBM),
        ],
        out_specs=pl.BlockSpec((1, blk, feat), lambda b, j: (b, j + lo, 0)),
        out_shape=jax.ShapeDtypeStruct((nb, seq, feat), grad_last.dtype),
        input_output_aliases={1: 0},
        compiler_params=pltpu.CompilerParams(
            dimension_semantics=("parallel", "arbitrary"),
        ),
    )(g, padded)
    return out.reshape(b0, b1, seq, feat)
